# baseline (speedup 1.0000x reference)
import functools

import jax
import jax.numpy as jnp
from jax import lax
from jax.experimental import pallas as pl
from jax.experimental.pallas import tpu as pltpu
from jax.experimental.pallas import tpu_sc as plsc

R = 4


def _emb_kernel(rows_per_w, hist, embed, nc, day_hbm, table_hbm, out_hbm,
                tshared, idx_v, rows_v, isem, gsem, osem):
    wid = lax.axis_index("s") * nc + lax.axis_index("c")
    n_chunks = rows_per_w // R
    chunk = R * hist
    w_day = wid * rows_per_w
    w_out = wid * rows_per_w * hist

    @pl.when(lax.axis_index("s") == 0)
    def _():
        pltpu.sync_copy(table_hbm, tshared)

    plsc.subcore_barrier()

    def fire_idx(c):
        pltpu.async_copy(
            day_hbm.at[pl.ds(w_out + c * chunk, chunk)],
            idx_v.at[lax.rem(c, 3)], isem)

    def drain_idx(c):
        pltpu.make_async_copy(
            day_hbm.at[pl.ds(0, chunk)], idx_v.at[lax.rem(c, 3)], isem).wait()

    def fire_gathers(c):
        pltpu.async_copy(
            tshared.at[idx_v.at[lax.rem(c, 3)]], rows_v.at[lax.rem(c, 2)],
            gsem)

    def drain_gathers(c):
        pltpu.make_async_copy(
            out_hbm.at[pl.ds(0, chunk)], rows_v.at[lax.rem(c, 2)],
            gsem).wait()

    def fire_write(c):
        pltpu.async_copy(
            rows_v.at[lax.rem(c, 2)],
            out_hbm.at[pl.ds(w_out + c * chunk, chunk)], osem)

    def drain_write(c):
        pltpu.make_async_copy(
            rows_v.at[lax.rem(c, 2)], out_hbm.at[pl.ds(0, chunk)],
            osem).wait()

    fire_idx(0)
    fire_idx(1)
    drain_idx(0)
    fire_gathers(0)

    def body(c, carry):
        @pl.when(c + 2 < n_chunks)
        def _():
            fire_idx(c + 2)

        @pl.when(c >= 1)
        def _():
            drain_write(c - 1)

        @pl.when(c + 1 < n_chunks)
        def _():
            drain_idx(c + 1)
            fire_gathers(c + 1)

        drain_gathers(c)
        fire_write(c)
        return carry

    lax.fori_loop(0, n_chunks, body, 0)
    drain_write(n_chunks - 1)


def kernel(day, table):
    batch, hist = day.shape
    vocab, embed = table.shape
    n = batch * hist

    info = plsc.get_sparse_core_info()
    nc, ns = info.num_cores, info.num_subcores
    nw = nc * ns
    assert batch % (nw * R) == 0
    rows_per_w = batch // nw
    chunk = R * hist

    mesh = plsc.VectorSubcoreMesh(core_axis_name="c", subcore_axis_name="s")
    k = functools.partial(
        pl.kernel,
        mesh=mesh,
        out_type=jax.ShapeDtypeStruct((n, embed), jnp.float32),
        scratch_types=[
            pltpu.VMEM_SHARED((vocab, embed), jnp.float32),
            pltpu.VMEM((3, chunk), jnp.int32),
            pltpu.VMEM((2, chunk, embed), jnp.float32),
            pltpu.SemaphoreType.DMA,
            pltpu.SemaphoreType.DMA,
            pltpu.SemaphoreType.DMA,
        ],
        compiler_params=pltpu.CompilerParams(use_tc_tiling_on_sc=False),
    )(functools.partial(_emb_kernel, rows_per_w, hist, embed, nc))

    z = lax.optimization_barrier(jnp.zeros((), jnp.int32))
    flat = k(day.reshape(n) + z, table)
    return flat.reshape(batch, hist, embed)


# speedup vs baseline: 1.0005x; 1.0005x over previous
import functools

import jax
import jax.numpy as jnp
from jax import lax
from jax.experimental import pallas as pl
from jax.experimental.pallas import tpu as pltpu
from jax.experimental.pallas import tpu_sc as plsc

R = 4


def _emb_kernel(rows_per_w, hist, embed, nc, day_hbm, table_hbm, out_hbm,
                tshared, idx_v, rows_v, isem, gsem, osem):
    wid = lax.axis_index("s") * nc + lax.axis_index("c")
    n_chunks = rows_per_w // R
    w_day = wid * rows_per_w

    @pl.when(lax.axis_index("s") == 0)
    def _():
        pltpu.sync_copy(table_hbm, tshared)

    plsc.subcore_barrier()

    def fire_idx(c):
        pltpu.async_copy(
            day_hbm.at[pl.ds(w_day + c * R, R)], idx_v.at[lax.rem(c, 3)],
            isem)

    def drain_idx(c):
        pltpu.make_async_copy(
            day_hbm.at[pl.ds(0, R)], idx_v.at[lax.rem(c, 3)], isem).wait()

    def fire_gathers(c):
        b = lax.rem(c, 2)
        b3 = lax.rem(c, 3)
        for r in range(R):
            pltpu.async_copy(
                tshared.at[idx_v.at[b3, r]], rows_v.at[b, r], gsem)

    def drain_gathers(c):
        pltpu.make_async_copy(
            out_hbm.at[pl.ds(0, R)], rows_v.at[lax.rem(c, 2)], gsem).wait()

    def fire_write(c):
        pltpu.async_copy(
            rows_v.at[lax.rem(c, 2)],
            out_hbm.at[pl.ds(w_day + c * R, R)], osem)

    def drain_write(c):
        pltpu.make_async_copy(
            rows_v.at[lax.rem(c, 2)], out_hbm.at[pl.ds(0, R)], osem).wait()

    fire_idx(0)
    fire_idx(1)
    drain_idx(0)
    fire_gathers(0)

    def body(c, carry):
        @pl.when(c + 2 < n_chunks)
        def _():
            fire_idx(c + 2)

        @pl.when(c >= 1)
        def _():
            drain_write(c - 1)

        @pl.when(c + 1 < n_chunks)
        def _():
            drain_idx(c + 1)
            fire_gathers(c + 1)

        drain_gathers(c)
        fire_write(c)
        return carry

    lax.fori_loop(0, n_chunks, body, 0)
    drain_write(n_chunks - 1)


def kernel(day, table):
    batch, hist = day.shape
    vocab, embed = table.shape

    info = plsc.get_sparse_core_info()
    nc, ns = info.num_cores, info.num_subcores
    nw = nc * ns
    assert batch % (nw * R) == 0
    rows_per_w = batch // nw

    mesh = plsc.VectorSubcoreMesh(core_axis_name="c", subcore_axis_name="s")
    k = functools.partial(
        pl.kernel,
        mesh=mesh,
        out_type=jax.ShapeDtypeStruct((batch, hist, embed), jnp.float32),
        scratch_types=[
            pltpu.VMEM_SHARED((vocab, embed), jnp.float32),
            pltpu.VMEM((3, R, hist), jnp.int32),
            pltpu.VMEM((2, R, hist, embed), jnp.float32),
            pltpu.SemaphoreType.DMA,
            pltpu.SemaphoreType.DMA,
            pltpu.SemaphoreType.DMA,
        ],
        compiler_params=pltpu.CompilerParams(use_tc_tiling_on_sc=False),
    )(functools.partial(_emb_kernel, rows_per_w, hist, embed, nc))

    return k(day, table)
